# baseline (speedup 1.0000x reference)
import functools

import jax
import jax.numpy as jnp
from jax import lax
from jax.experimental import pallas as pl
from jax.experimental.pallas import tpu as pltpu
from jax.experimental.pallas import tpu_sc as plsc

B, S, D = 2, 2048, 768
E, ED, OD = 8, 1024, 768
T = B * S

BLK = 128
NB = T // BLK + E
CAP = NB * BLK
NBPAD = 64
L16 = 16
NC, NS = 2, 16
NW = NC * NS
RPW = CAP // NW
CHUNKS = 4
CR = RPW // CHUNKS
TPT = T // NS
SPT = CAP // NS
D2 = D // 2
OD2 = OD // 2


@functools.cache
def _sc_mesh():
    return plsc.VectorSubcoreMesh(
        core_axis_name="c", subcore_axis_name="s",
        num_cores=NC, num_subcores=NS)


def _gate_body(x_ref, gw_ref, gb_ref, out_ref):
    logits = lax.dot_general(
        x_ref[...], gw_ref[...], (((1,), (1,)), ((), ())),
        preferred_element_type=jnp.float32)
    logits = logits + gb_ref[...]
    out_ref[...] = jnp.argmax(logits, axis=-1).astype(jnp.int32)


def _gate(xf, gate_W, gate_b):
    blk = 512
    return pl.pallas_call(
        _gate_body,
        grid=(T // blk,),
        in_specs=[
            pl.BlockSpec((blk, D), lambda i: (i, 0)),
            pl.BlockSpec((E, D), lambda i: (0, 0)),
            pl.BlockSpec((1, E), lambda i: (0, 0)),
        ],
        out_specs=pl.BlockSpec((blk,), lambda i: (i,)),
        out_shape=jax.ShapeDtypeStruct((T,), jnp.int32),
    )(xf, gate_W, gate_b.reshape(1, E))


def _route_body_v1(assign_hbm, perm_hbm, bexp_hbm,
                   assign_v, perm_v, bexp_v):
    cid = lax.axis_index("c")
    sid = lax.axis_index("s")

    @pl.when(jnp.logical_and(cid == 0, sid == 0))
    def _():
        pltpu.sync_copy(assign_hbm, assign_v)
        lanes = lax.iota(jnp.int32, L16)

        def init_body(j, carry):
            idx = j * L16 + lanes
            plsc.store_scatter(perm_v, [idx], jnp.full((L16,), T, jnp.int32))
            return carry

        lax.fori_loop(0, CAP // L16, init_body, jnp.int32(0))

        ends = []
        start = jnp.int32(0)
        for e in range(E):
            def scan_body(j, cnt, e=e, start=start):
                idx = j * L16 + lanes
                a = plsc.load_gather(assign_v, [idx])
                m = a == e
                mi = m.astype(jnp.int32)
                rank = plsc.cumsum(mi) - mi
                pos = start + cnt + rank
                plsc.store_scatter(perm_v, [pos], idx, mask=m)
                return cnt + jnp.sum(mi)

            cnt = lax.fori_loop(0, T // L16, scan_body, jnp.int32(0))
            nblk = (cnt + BLK - 1) // BLK
            start = start + nblk * BLK
            ends.append(start)

        for v in range(NBPAD // L16):
            bidx = v * L16 + lanes
            bstart = bidx * BLK
            eid = jnp.zeros((L16,), jnp.int32)
            for e in range(E - 1):
                eid = eid + (bstart >= ends[e]).astype(jnp.int32)
            plsc.store_scatter(bexp_v, [bidx], eid)

        pltpu.sync_copy(perm_v, perm_hbm)
        pltpu.sync_copy(bexp_v, bexp_hbm)


@functools.cache
def _route_v1():
    return pl.kernel(
        _route_body_v1,
        out_type=(
            jax.ShapeDtypeStruct((CAP,), jnp.int32),
            jax.ShapeDtypeStruct((NBPAD,), jnp.int32),
        ),
        mesh=_sc_mesh(),
        compiler_params=pltpu.CompilerParams(needs_layout_passes=False),
        scratch_types=[
            pltpu.VMEM((T,), jnp.int32),
            pltpu.VMEM((CAP,), jnp.int32),
            pltpu.VMEM((NBPAD,), jnp.int32),
        ],
    )


def _route_body(assign_hbm, perm_hbm, bexp_hbm,
                assign_v, fill_v, cnt_v, all_cnt_v, pos_v, val_v, bexp_v,
                sh_cnt):
    cid = lax.axis_index("c")
    sid = lax.axis_index("s")

    @pl.when(cid == 0)
    def _():
        lanes = lax.iota(jnp.int32, L16)
        base = sid * TPT
        pltpu.sync_copy(assign_hbm.at[pl.ds(base, TPT)], assign_v)

        fill = jnp.full((L16,), T, jnp.int32)
        for j in range(SPT // L16):
            plsc.store_scatter(fill_v, [j * L16 + lanes], fill)
        pltpu.sync_copy(fill_v, perm_hbm.at[pl.ds(sid * SPT, SPT)])

        avs = [plsc.load_gather(assign_v, [j * L16 + lanes])
               for j in range(TPT // L16)]
        cnts = []
        for e in range(E):
            c = jnp.int32(0)
            for a in avs:
                c = c + jnp.sum((a == e).astype(jnp.int32))
            cnts.append(c)
        cnt_vec = jnp.zeros((L16,), jnp.int32)
        for e in range(E):
            cnt_vec = jnp.where(lanes == e, cnts[e], cnt_vec)
        plsc.store_scatter(cnt_v, [lanes], cnt_vec)
        pltpu.sync_copy(cnt_v, sh_cnt.at[sid])
        plsc.subcore_barrier()

        pltpu.sync_copy(sh_cnt, all_cnt_v)
        acc = jnp.zeros((L16,), jnp.int32)
        pref = jnp.zeros((L16,), jnp.int32)
        for wp in range(NS):
            row = plsc.load_gather(
                all_cnt_v, [jnp.full((L16,), wp, jnp.int32), lanes])
            pref = jnp.where(jnp.full((L16,), wp, jnp.int32) == sid,
                             acc, pref)
            acc = acc + row
        padded = ((acc + BLK - 1) // BLK) * BLK
        pend = plsc.cumsum(padded)
        myoff = (pend - padded) + pref
        offs = [jnp.sum(jnp.where(lanes == e, myoff, 0))
                for e in range(E)]

        runs = [jnp.int32(0)] * E
        for j in range(TPT // L16):
            a = avs[j]
            tok = base + j * L16 + lanes
            pos = jnp.zeros((L16,), jnp.int32)
            for e in range(E):
                m = a == e
                mi = m.astype(jnp.int32)
                rank = plsc.cumsum(mi) - mi
                pos = jnp.where(m, offs[e] + runs[e] + rank, pos)
                runs[e] = runs[e] + jnp.sum(mi)
            half = j // 8
            slot = (j % 8) * L16 + lanes
            plsc.store_scatter(pos_v, [jnp.full((L16,), half, jnp.int32),
                                       slot], pos)
            plsc.store_scatter(val_v, [jnp.full((L16,), half, jnp.int32),
                                       slot], tok)
        for h in range(TPT // 128):
            pltpu.sync_copy(val_v.at[h], perm_hbm.at[pos_v.at[h]])

        @pl.when(sid == 0)
        def _():
            ends = [jnp.sum(jnp.where(lanes == e, pend, 0))
                    for e in range(E - 1)]
            for v in range(NBPAD // L16):
                bidx = v * L16 + lanes
                bstart = bidx * BLK
                eid = jnp.zeros((L16,), jnp.int32)
                for e in range(E - 1):
                    eid = eid + (bstart >= ends[e]).astype(jnp.int32)
                plsc.store_scatter(bexp_v, [bidx], eid)
            pltpu.sync_copy(bexp_v, bexp_hbm)


@functools.cache
def _route():
    return pl.kernel(
        _route_body,
        out_type=(
            jax.ShapeDtypeStruct((CAP,), jnp.int32),
            jax.ShapeDtypeStruct((NBPAD,), jnp.int32),
        ),
        mesh=_sc_mesh(),
        compiler_params=pltpu.CompilerParams(needs_layout_passes=False),
        scratch_types=[
            pltpu.VMEM((TPT,), jnp.int32),
            pltpu.VMEM((SPT,), jnp.int32),
            pltpu.VMEM((L16,), jnp.int32),
            pltpu.VMEM((NS, L16), jnp.int32),
            pltpu.VMEM((TPT // 128, 128), jnp.int32),
            pltpu.VMEM((TPT // 128, 128), jnp.int32),
            pltpu.VMEM((NBPAD,), jnp.int32),
            pltpu.VMEM_SHARED((NS, L16), jnp.int32),
        ],
    )


def _gather_body(x_hbm, perm2_hbm, xs_hbm, idx_v, rows_v, sem_i, sem_g,
                 sem_w):
    cid = lax.axis_index("c")
    sid = lax.axis_index("s")
    wid = sid * NC + cid
    lanes = lax.iota(jnp.int32, L16)

    ci = [pltpu.async_copy(perm2_hbm.at[wid * CHUNKS + c], idx_v.at[c],
                           sem_i.at[c])
          for c in range(CHUNKS)]
    cg = [None] * CHUNKS
    cw = [None] * CHUNKS
    for c in range(CHUNKS):
        ci[c].wait()
        for j in range(CR // L16):
            ii = j * L16 + lanes
            cc = jnp.full((L16,), c, jnp.int32)
            v = plsc.load_gather(idx_v, [cc, ii])
            plsc.store_scatter(idx_v, [cc, ii], jnp.minimum(v, T - 1))
        cg[c] = pltpu.async_copy(x_hbm.at[idx_v.at[c]], rows_v.at[c],
                                 sem_g.at[c])
    for c in range(CHUNKS):
        cg[c].wait()
        cw[c] = pltpu.async_copy(rows_v.at[c], xs_hbm.at[wid, c],
                                 sem_w.at[c])
    for c in range(CHUNKS):
        cw[c].wait()


@functools.cache
def _gather():
    return pl.kernel(
        _gather_body,
        out_type=jax.ShapeDtypeStruct((NW, CHUNKS, CR, D2), jnp.int32),
        mesh=_sc_mesh(),
        compiler_params=pltpu.CompilerParams(needs_layout_passes=False),
        scratch_types=[
            pltpu.VMEM((CHUNKS, CR), jnp.int32),
            pltpu.VMEM((CHUNKS, CR, D2), jnp.int32),
            pltpu.SemaphoreType.DMA((CHUNKS,)),
            pltpu.SemaphoreType.DMA((CHUNKS,)),
            pltpu.SemaphoreType.DMA((CHUNKS,)),
        ],
    )


def _mm_body(bexp_ref, xs_ref, ew_ref, eb_ref, ow_ref, ob_ref, out_ref):
    h = lax.dot_general(
        xs_ref[...], ew_ref[0], (((1,), (1,)), ((), ())),
        preferred_element_type=jnp.float32)
    h = (h + eb_ref[0]).astype(jnp.bfloat16)
    o = lax.dot_general(
        h, ow_ref[...], (((1,), (1,)), ((), ())),
        preferred_element_type=jnp.float32)
    out_ref[...] = (o + ob_ref[...]).astype(jnp.bfloat16)


def _mm(bexp, xs, expert_W, expert_b, out_W, out_b):
    return pl.pallas_call(
        _mm_body,
        grid_spec=pltpu.PrefetchScalarGridSpec(
            num_scalar_prefetch=1,
            grid=(NB,),
            in_specs=[
                pl.BlockSpec((BLK, D), lambda i, b: (i, 0)),
                pl.BlockSpec((1, ED, D), lambda i, b: (b[i], 0, 0)),
                pl.BlockSpec((1, 1, ED), lambda i, b: (b[i], 0, 0)),
                pl.BlockSpec((OD, ED), lambda i, b: (0, 0)),
                pl.BlockSpec((1, OD), lambda i, b: (0, 0)),
            ],
            out_specs=pl.BlockSpec((BLK, OD), lambda i, b: (i, 0)),
        ),
        out_shape=jax.ShapeDtypeStruct((CAP, OD), jnp.bfloat16),
        compiler_params=pltpu.CompilerParams(
            dimension_semantics=("arbitrary",)),
    )(bexp, xs, expert_W, expert_b.reshape(E, 1, ED), out_W,
      out_b.reshape(1, OD))


def _scatter_body(ys_hbm, perm2_hbm, opad_hbm, idx_v, rows_v, sem_i, sem_r,
                  sem_s):
    cid = lax.axis_index("c")
    sid = lax.axis_index("s")
    wid = sid * NC + cid

    ci = [pltpu.async_copy(perm2_hbm.at[wid * CHUNKS + c], idx_v.at[c],
                           sem_i.at[c])
          for c in range(CHUNKS)]
    cr = [pltpu.async_copy(ys_hbm.at[wid, c], rows_v.at[c], sem_r.at[c])
          for c in range(CHUNKS)]
    cs = [None] * CHUNKS
    for c in range(CHUNKS):
        ci[c].wait()
        cr[c].wait()
        cs[c] = pltpu.async_copy(rows_v.at[c], opad_hbm.at[idx_v.at[c]],
                                 sem_s.at[c])
    for c in range(CHUNKS):
        cs[c].wait()


@functools.cache
def _scatter():
    return pl.kernel(
        _scatter_body,
        out_type=jax.ShapeDtypeStruct((T + 8, OD2), jnp.int32),
        mesh=_sc_mesh(),
        compiler_params=pltpu.CompilerParams(needs_layout_passes=False),
        scratch_types=[
            pltpu.VMEM((CHUNKS, CR), jnp.int32),
            pltpu.VMEM((CHUNKS, CR, OD2), jnp.int32),
            pltpu.SemaphoreType.DMA((CHUNKS,)),
            pltpu.SemaphoreType.DMA((CHUNKS,)),
            pltpu.SemaphoreType.DMA((CHUNKS,)),
        ],
    )


def kernel(x, gate_W, gate_b, expert_W, expert_b, out_W, out_b):
    assert x.shape == (B, S, D)
    assert expert_W.shape == (E, ED, D)
    assert out_W.shape == (OD, ED)

    xf = x.reshape(T, D)
    xbf = xf.astype(jnp.bfloat16)
    ewbf = expert_W.astype(jnp.bfloat16)
    owbf = out_W.astype(jnp.bfloat16)
    xp = lax.bitcast_convert_type(xbf.reshape(T, D2, 2), jnp.int32)
    assign = _gate(xf, gate_W, gate_b)
    perm, bexp = _route_v1()(assign)
    perm2 = perm.reshape(NW * CHUNKS, CR)
    xs = _gather()(xp, perm2)
    xsbf = lax.bitcast_convert_type(
        xs.reshape(CAP, D2), jnp.bfloat16).reshape(CAP, D)
    ys = _mm(bexp, xsbf, ewbf, expert_b, owbf, out_b)
    ysp = lax.bitcast_convert_type(
        ys.reshape(CAP, OD2, 2), jnp.int32).reshape(NW, CHUNKS, CR, OD2)
    opad = _scatter()(ysp, perm2)
    obf = lax.bitcast_convert_type(opad[:T], jnp.bfloat16).reshape(T, OD)
    return obf.astype(jnp.float32).reshape(B, S, OD)


# speedup vs baseline: 2.4105x; 2.4105x over previous
import functools

import jax
import jax.numpy as jnp
from jax import lax
from jax.experimental import pallas as pl
from jax.experimental.pallas import tpu as pltpu
from jax.experimental.pallas import tpu_sc as plsc

B, S, D = 2, 2048, 768
E, ED, OD = 8, 1024, 768
T = B * S

BLK = 128
NB = T // BLK + E
CAP = NB * BLK
NBPAD = 64
L16 = 16
NC, NS = 2, 16
NW = NC * NS
RPW = CAP // NW
CHUNKS = 4
CR = RPW // CHUNKS
TPT = T // NS
SPT = CAP // NS
D2 = D // 2
OD2 = OD // 2


@functools.cache
def _sc_mesh():
    return plsc.VectorSubcoreMesh(
        core_axis_name="c", subcore_axis_name="s",
        num_cores=NC, num_subcores=NS)


def _bf16_bits(v):
    b = lax.bitcast_convert_type(v, jnp.int32)
    return b + 0x7FFF + (lax.shift_right_logical(b, 16) & 1)


def _pack_pair(lo, hi):
    return (_bf16_bits(hi) & jnp.int32(-65536)) | lax.shift_right_logical(
        _bf16_bits(lo), 16)


def _unpack_lo(w):
    return lax.bitcast_convert_type(lax.shift_left(w, 16), jnp.float32)


def _unpack_hi(w):
    return lax.bitcast_convert_type(w & jnp.int32(-65536), jnp.float32)


def _gate_body(x_ref, gw_ref, gb_ref, out_ref, xp_ref):
    xv = x_ref[...]
    logits = lax.dot_general(
        xv, gw_ref[...], (((1,), (1,)), ((), ())),
        preferred_element_type=jnp.float32)
    logits = logits + gb_ref[...]
    out_ref[...] = jnp.argmax(logits, axis=-1).astype(jnp.int32)
    xp_ref[...] = _pack_pair(xv[:, :D2], xv[:, D2:])


def _gate(xf, gate_W, gate_b):
    blk = 512
    return pl.pallas_call(
        _gate_body,
        grid=(T // blk,),
        in_specs=[
            pl.BlockSpec((blk, D), lambda i: (i, 0)),
            pl.BlockSpec((E, D), lambda i: (0, 0)),
            pl.BlockSpec((1, E), lambda i: (0, 0)),
        ],
        out_specs=[
            pl.BlockSpec((blk,), lambda i: (i,)),
            pl.BlockSpec((blk, D2), lambda i: (i, 0)),
        ],
        out_shape=[
            jax.ShapeDtypeStruct((T,), jnp.int32),
            jax.ShapeDtypeStruct((T, D2), jnp.int32),
        ],
    )(xf, gate_W, gate_b.reshape(1, E))


def _route_body_v1(assign_hbm, perm_hbm, bexp_hbm,
                   assign_v, perm_v, bexp_v):
    cid = lax.axis_index("c")
    sid = lax.axis_index("s")

    @pl.when(jnp.logical_and(cid == 0, sid == 0))
    def _():
        pltpu.sync_copy(assign_hbm, assign_v)
        lanes = lax.iota(jnp.int32, L16)

        def init_body(j, carry):
            idx = j * L16 + lanes
            plsc.store_scatter(perm_v, [idx], jnp.full((L16,), T, jnp.int32))
            return carry

        lax.fori_loop(0, CAP // L16, init_body, jnp.int32(0))

        ends = []
        start = jnp.int32(0)
        for e in range(E):
            def scan_body(j, cnt, e=e, start=start):
                idx = j * L16 + lanes
                a = plsc.load_gather(assign_v, [idx])
                m = a == e
                mi = m.astype(jnp.int32)
                rank = plsc.cumsum(mi) - mi
                pos = start + cnt + rank
                plsc.store_scatter(perm_v, [pos], idx, mask=m)
                return cnt + jnp.sum(mi)

            cnt = lax.fori_loop(0, T // L16, scan_body, jnp.int32(0))
            nblk = (cnt + BLK - 1) // BLK
            start = start + nblk * BLK
            ends.append(start)

        for v in range(NBPAD // L16):
            bidx = v * L16 + lanes
            bstart = bidx * BLK
            eid = jnp.zeros((L16,), jnp.int32)
            for e in range(E - 1):
                eid = eid + (bstart >= ends[e]).astype(jnp.int32)
            plsc.store_scatter(bexp_v, [bidx], eid)

        pltpu.sync_copy(perm_v, perm_hbm)
        pltpu.sync_copy(bexp_v, bexp_hbm)


@functools.cache
def _route_v1():
    return pl.kernel(
        _route_body_v1,
        out_type=(
            jax.ShapeDtypeStruct((CAP,), jnp.int32),
            jax.ShapeDtypeStruct((NBPAD,), jnp.int32),
        ),
        mesh=_sc_mesh(),
        compiler_params=pltpu.CompilerParams(needs_layout_passes=False),
        scratch_types=[
            pltpu.VMEM((T,), jnp.int32),
            pltpu.VMEM((CAP,), jnp.int32),
            pltpu.VMEM((NBPAD,), jnp.int32),
        ],
    )


def _route_body(assign_hbm, perm_hbm, bexp_hbm,
                assign_v, fill_v, cnt_v, all_cnt_v, pos_v, val_v, bexp_v,
                sh_cnt):
    cid = lax.axis_index("c")
    sid = lax.axis_index("s")

    @pl.when(cid == 0)
    def _():
        lanes = lax.iota(jnp.int32, L16)
        base = sid * TPT
        pltpu.sync_copy(assign_hbm.at[pl.ds(base, TPT)], assign_v)

        fill = jnp.full((L16,), T, jnp.int32)
        for j in range(SPT // L16):
            plsc.store_scatter(fill_v, [j * L16 + lanes], fill)
        pltpu.sync_copy(fill_v, perm_hbm.at[pl.ds(sid * SPT, SPT)])

        avs = [plsc.load_gather(assign_v, [j * L16 + lanes])
               for j in range(TPT // L16)]
        cnts = []
        for e in range(E):
            c = jnp.int32(0)
            for a in avs:
                c = c + jnp.sum((a == e).astype(jnp.int32))
            cnts.append(c)
        cnt_vec = jnp.zeros((L16,), jnp.int32)
        for e in range(E):
            cnt_vec = jnp.where(lanes == e, cnts[e], cnt_vec)
        plsc.store_scatter(cnt_v, [lanes], cnt_vec)
        pltpu.sync_copy(cnt_v, sh_cnt.at[sid])
        plsc.subcore_barrier()

        pltpu.sync_copy(sh_cnt, all_cnt_v)
        acc = jnp.zeros((L16,), jnp.int32)
        pref = jnp.zeros((L16,), jnp.int32)
        for wp in range(NS):
            row = plsc.load_gather(
                all_cnt_v, [jnp.full((L16,), wp, jnp.int32), lanes])
            pref = jnp.where(jnp.full((L16,), wp, jnp.int32) == sid,
                             acc, pref)
            acc = acc + row
        padded = ((acc + BLK - 1) // BLK) * BLK
        pend = plsc.cumsum(padded)
        myoff = (pend - padded) + pref
        offs = [jnp.sum(jnp.where(lanes == e, myoff, 0))
                for e in range(E)]

        runs = [jnp.int32(0)] * E
        for j in range(TPT // L16):
            a = avs[j]
            tok = base + j * L16 + lanes
            pos = jnp.zeros((L16,), jnp.int32)
            for e in range(E):
                m = a == e
                mi = m.astype(jnp.int32)
                rank = plsc.cumsum(mi) - mi
                pos = jnp.where(m, offs[e] + runs[e] + rank, pos)
                runs[e] = runs[e] + jnp.sum(mi)
            half = j // 8
            slot = (j % 8) * L16 + lanes
            plsc.store_scatter(pos_v, [jnp.full((L16,), half, jnp.int32),
                                       slot], pos)
            plsc.store_scatter(val_v, [jnp.full((L16,), half, jnp.int32),
                                       slot], tok)
        for h in range(TPT // 128):
            pltpu.sync_copy(val_v.at[h], perm_hbm.at[pos_v.at[h]])

        @pl.when(sid == 0)
        def _():
            ends = [jnp.sum(jnp.where(lanes == e, pend, 0))
                    for e in range(E - 1)]
            for v in range(NBPAD // L16):
                bidx = v * L16 + lanes
                bstart = bidx * BLK
                eid = jnp.zeros((L16,), jnp.int32)
                for e in range(E - 1):
                    eid = eid + (bstart >= ends[e]).astype(jnp.int32)
                plsc.store_scatter(bexp_v, [bidx], eid)
            pltpu.sync_copy(bexp_v, bexp_hbm)


@functools.cache
def _route():
    return pl.kernel(
        _route_body,
        out_type=(
            jax.ShapeDtypeStruct((CAP,), jnp.int32),
            jax.ShapeDtypeStruct((NBPAD,), jnp.int32),
        ),
        mesh=_sc_mesh(),
        compiler_params=pltpu.CompilerParams(needs_layout_passes=False),
        scratch_types=[
            pltpu.VMEM((TPT,), jnp.int32),
            pltpu.VMEM((SPT,), jnp.int32),
            pltpu.VMEM((L16,), jnp.int32),
            pltpu.VMEM((NS, L16), jnp.int32),
            pltpu.VMEM((TPT // 128, 128), jnp.int32),
            pltpu.VMEM((TPT // 128, 128), jnp.int32),
            pltpu.VMEM((NBPAD,), jnp.int32),
            pltpu.VMEM_SHARED((NS, L16), jnp.int32),
        ],
    )


def _gather_body(x_hbm, perm2_hbm, xs_hbm, idx_v, rows_v, sem_i, sem_g,
                 sem_w):
    cid = lax.axis_index("c")
    sid = lax.axis_index("s")
    wid = sid * NC + cid
    lanes = lax.iota(jnp.int32, L16)

    ci = [pltpu.async_copy(perm2_hbm.at[wid * CHUNKS + c], idx_v.at[c],
                           sem_i.at[c])
          for c in range(CHUNKS)]
    cg = [None] * CHUNKS
    cw = [None] * CHUNKS
    for c in range(CHUNKS):
        ci[c].wait()
        for j in range(CR // L16):
            ii = j * L16 + lanes
            cc = jnp.full((L16,), c, jnp.int32)
            v = plsc.load_gather(idx_v, [cc, ii])
            plsc.store_scatter(idx_v, [cc, ii], jnp.minimum(v, T - 1))
        cg[c] = pltpu.async_copy(x_hbm.at[idx_v.at[c]], rows_v.at[c],
                                 sem_g.at[c])
    for c in range(CHUNKS):
        cg[c].wait()
        cw[c] = pltpu.async_copy(rows_v.at[c], xs_hbm.at[wid, c],
                                 sem_w.at[c])
    for c in range(CHUNKS):
        cw[c].wait()


@functools.cache
def _gather():
    return pl.kernel(
        _gather_body,
        out_type=jax.ShapeDtypeStruct((NW, CHUNKS, CR, D2), jnp.int32),
        mesh=_sc_mesh(),
        compiler_params=pltpu.CompilerParams(needs_layout_passes=False),
        scratch_types=[
            pltpu.VMEM((CHUNKS, CR), jnp.int32),
            pltpu.VMEM((CHUNKS, CR, D2), jnp.int32),
            pltpu.SemaphoreType.DMA((CHUNKS,)),
            pltpu.SemaphoreType.DMA((CHUNKS,)),
            pltpu.SemaphoreType.DMA((CHUNKS,)),
        ],
    )


def _mm_body(bexp_ref, xs_ref, ew_ref, eb_ref, ow_ref, ob_ref, out_ref):
    w = xs_ref[...]
    xb = jnp.concatenate(
        [_unpack_lo(w), _unpack_hi(w)], axis=1).astype(jnp.bfloat16)
    h = lax.dot_general(
        xb, ew_ref[0], (((1,), (1,)), ((), ())),
        preferred_element_type=jnp.float32)
    h = (h + eb_ref[0]).astype(jnp.bfloat16)
    o = lax.dot_general(
        h, ow_ref[...], (((1,), (1,)), ((), ())),
        preferred_element_type=jnp.float32)
    o = o + ob_ref[...]
    out_ref[...] = _pack_pair(o[:, :OD2], o[:, OD2:])


def _mm(bexp, xs, expert_W, expert_b, out_W, out_b):
    return pl.pallas_call(
        _mm_body,
        grid_spec=pltpu.PrefetchScalarGridSpec(
            num_scalar_prefetch=1,
            grid=(NB,),
            in_specs=[
                pl.BlockSpec((BLK, D2), lambda i, b: (i, 0)),
                pl.BlockSpec((1, ED, D), lambda i, b: (b[i], 0, 0)),
                pl.BlockSpec((1, 1, ED), lambda i, b: (b[i], 0, 0)),
                pl.BlockSpec((OD, ED), lambda i, b: (0, 0)),
                pl.BlockSpec((1, OD), lambda i, b: (0, 0)),
            ],
            out_specs=pl.BlockSpec((BLK, OD2), lambda i, b: (i, 0)),
        ),
        out_shape=jax.ShapeDtypeStruct((CAP, OD2), jnp.int32),
        compiler_params=pltpu.CompilerParams(
            dimension_semantics=("arbitrary",)),
    )(bexp, xs, expert_W, expert_b.reshape(E, 1, ED), out_W,
      out_b.reshape(1, OD))


def _unpack_body(op_ref, out_ref):
    w = op_ref[...]
    out_ref[...] = jnp.concatenate([_unpack_lo(w), _unpack_hi(w)], axis=1)


def _unpack_out(opad):
    blk = 512
    return pl.pallas_call(
        _unpack_body,
        grid=(T // blk,),
        in_specs=[pl.BlockSpec((blk, OD2), lambda i: (i, 0))],
        out_specs=pl.BlockSpec((blk, OD), lambda i: (i, 0)),
        out_shape=jax.ShapeDtypeStruct((T, OD), jnp.float32),
    )(opad)


def _scatter_body(ys_hbm, perm2_hbm, opad_hbm, idx_v, rows_v, sem_i, sem_r,
                  sem_s):
    cid = lax.axis_index("c")
    sid = lax.axis_index("s")
    wid = sid * NC + cid

    ci = [pltpu.async_copy(perm2_hbm.at[wid * CHUNKS + c], idx_v.at[c],
                           sem_i.at[c])
          for c in range(CHUNKS)]
    cr = [pltpu.async_copy(ys_hbm.at[wid, c], rows_v.at[c], sem_r.at[c])
          for c in range(CHUNKS)]
    cs = [None] * CHUNKS
    for c in range(CHUNKS):
        ci[c].wait()
        cr[c].wait()
        cs[c] = pltpu.async_copy(rows_v.at[c], opad_hbm.at[idx_v.at[c]],
                                 sem_s.at[c])
    for c in range(CHUNKS):
        cs[c].wait()


@functools.cache
def _scatter():
    return pl.kernel(
        _scatter_body,
        out_type=jax.ShapeDtypeStruct((T + 8, OD2), jnp.int32),
        mesh=_sc_mesh(),
        compiler_params=pltpu.CompilerParams(needs_layout_passes=False),
        scratch_types=[
            pltpu.VMEM((CHUNKS, CR), jnp.int32),
            pltpu.VMEM((CHUNKS, CR, OD2), jnp.int32),
            pltpu.SemaphoreType.DMA((CHUNKS,)),
            pltpu.SemaphoreType.DMA((CHUNKS,)),
            pltpu.SemaphoreType.DMA((CHUNKS,)),
        ],
    )


def kernel(x, gate_W, gate_b, expert_W, expert_b, out_W, out_b):
    assert x.shape == (B, S, D)
    assert expert_W.shape == (E, ED, D)
    assert out_W.shape == (OD, ED)

    xf = x.reshape(T, D)
    ewbf = expert_W.astype(jnp.bfloat16)
    owbf = out_W.astype(jnp.bfloat16)
    assign, xp = _gate(xf, gate_W, gate_b)
    perm, bexp = _route_v1()(assign)
    perm2 = perm.reshape(NW * CHUNKS, CR)
    xs = _gather()(xp, perm2)
    ys = _mm(bexp, xs.reshape(CAP, D2), ewbf, expert_b, owbf, out_b)
    opad = _scatter()(ys.reshape(NW, CHUNKS, CR, OD2), perm2)
    return _unpack_out(opad).reshape(B, S, OD)


# speedup vs baseline: 2.4543x; 1.0182x over previous
import functools

import jax
import jax.numpy as jnp
from jax import lax
from jax.experimental import pallas as pl
from jax.experimental.pallas import tpu as pltpu
from jax.experimental.pallas import tpu_sc as plsc

B, S, D = 2, 2048, 768
E, ED, OD = 8, 1024, 768
T = B * S

BLK = 128
NB = T // BLK + E
CAP = NB * BLK
NBPAD = 64
L16 = 16
NC, NS = 2, 16
NW = NC * NS
RPW = CAP // NW
CHUNKS = 4
CR = RPW // CHUNKS
TPT = T // NS
SPT = CAP // NS
D2 = D // 2
OD2 = OD // 2


@functools.cache
def _sc_mesh():
    return plsc.VectorSubcoreMesh(
        core_axis_name="c", subcore_axis_name="s",
        num_cores=NC, num_subcores=NS)


def _bf16_bits(v):
    b = lax.bitcast_convert_type(v, jnp.int32)
    return b + 0x7FFF + (lax.shift_right_logical(b, 16) & 1)


def _pack_pair(lo, hi):
    return (_bf16_bits(hi) & jnp.int32(-65536)) | lax.shift_right_logical(
        _bf16_bits(lo), 16)


def _unpack_lo(w):
    return lax.bitcast_convert_type(lax.shift_left(w, 16), jnp.float32)


def _unpack_hi(w):
    return lax.bitcast_convert_type(w & jnp.int32(-65536), jnp.float32)


def _gate_body(x_ref, gw_ref, gb_ref, out_ref, xp_ref):
    xv = x_ref[...]
    logits = lax.dot_general(
        xv, gw_ref[...], (((1,), (1,)), ((), ())),
        preferred_element_type=jnp.float32)
    logits = logits + gb_ref[...]
    out_ref[...] = jnp.argmax(logits, axis=-1).astype(jnp.int32)
    xp_ref[...] = _pack_pair(xv[:, :D2], xv[:, D2:])


def _gate(xf, gate_W, gate_b):
    blk = 512
    return pl.pallas_call(
        _gate_body,
        grid=(T // blk,),
        in_specs=[
            pl.BlockSpec((blk, D), lambda i: (i, 0)),
            pl.BlockSpec((E, D), lambda i: (0, 0)),
            pl.BlockSpec((1, E), lambda i: (0, 0)),
        ],
        out_specs=[
            pl.BlockSpec((blk,), lambda i: (i,)),
            pl.BlockSpec((blk, D2), lambda i: (i, 0)),
        ],
        out_shape=[
            jax.ShapeDtypeStruct((T,), jnp.int32),
            jax.ShapeDtypeStruct((T, D2), jnp.int32),
        ],
    )(xf, gate_W, gate_b.reshape(1, E))


def _route_body_v1(assign_hbm, perm_hbm, bexp_hbm,
                   assign_v, perm_v, bexp_v):
    cid = lax.axis_index("c")
    sid = lax.axis_index("s")

    @pl.when(jnp.logical_and(cid == 0, sid == 0))
    def _():
        pltpu.sync_copy(assign_hbm, assign_v)
        lanes = lax.iota(jnp.int32, L16)

        def init_body(j, carry):
            idx = j * L16 + lanes
            plsc.store_scatter(perm_v, [idx], jnp.full((L16,), T, jnp.int32))
            return carry

        lax.fori_loop(0, CAP // L16, init_body, jnp.int32(0))

        ends = []
        start = jnp.int32(0)
        for e in range(E):
            def scan_body(j, cnt, e=e, start=start):
                idx = j * L16 + lanes
                a = plsc.load_gather(assign_v, [idx])
                m = a == e
                mi = m.astype(jnp.int32)
                rank = plsc.cumsum(mi) - mi
                pos = start + cnt + rank
                plsc.store_scatter(perm_v, [pos], idx, mask=m)
                return cnt + jnp.sum(mi)

            cnt = lax.fori_loop(0, T // L16, scan_body, jnp.int32(0))
            nblk = (cnt + BLK - 1) // BLK
            start = start + nblk * BLK
            ends.append(start)

        for v in range(NBPAD // L16):
            bidx = v * L16 + lanes
            bstart = bidx * BLK
            eid = jnp.zeros((L16,), jnp.int32)
            for e in range(E - 1):
                eid = eid + (bstart >= ends[e]).astype(jnp.int32)
            plsc.store_scatter(bexp_v, [bidx], eid)

        pltpu.sync_copy(perm_v, perm_hbm)
        pltpu.sync_copy(bexp_v, bexp_hbm)


@functools.cache
def _route_v1():
    return pl.kernel(
        _route_body_v1,
        out_type=(
            jax.ShapeDtypeStruct((CAP,), jnp.int32),
            jax.ShapeDtypeStruct((NBPAD,), jnp.int32),
        ),
        mesh=_sc_mesh(),
        compiler_params=pltpu.CompilerParams(needs_layout_passes=False),
        scratch_types=[
            pltpu.VMEM((T,), jnp.int32),
            pltpu.VMEM((CAP,), jnp.int32),
            pltpu.VMEM((NBPAD,), jnp.int32),
        ],
    )


def _route_body(assign_hbm, perm_hbm, bexp_hbm,
                assign_v, fill_v, cnt_v, all_cnt_v, pos_v, val_v, bexp_v,
                sh_cnt):
    cid = lax.axis_index("c")
    sid = lax.axis_index("s")

    @pl.when(cid == 0)
    def _():
        lanes = lax.iota(jnp.int32, L16)
        base = sid * TPT
        pltpu.sync_copy(assign_hbm.at[pl.ds(base, TPT)], assign_v)

        fill = jnp.full((L16,), T, jnp.int32)
        for j in range(SPT // L16):
            plsc.store_scatter(fill_v, [j * L16 + lanes], fill)
        pltpu.sync_copy(fill_v, perm_hbm.at[pl.ds(sid * SPT, SPT)])

        avs = [plsc.load_gather(assign_v, [j * L16 + lanes])
               for j in range(TPT // L16)]
        cnts = []
        for e in range(E):
            c = jnp.int32(0)
            for a in avs:
                c = c + jnp.sum((a == e).astype(jnp.int32))
            cnts.append(c)
        cnt_vec = jnp.zeros((L16,), jnp.int32)
        for e in range(E):
            cnt_vec = jnp.where(lanes == e, cnts[e], cnt_vec)
        plsc.store_scatter(cnt_v, [lanes], cnt_vec)
        pltpu.sync_copy(cnt_v, sh_cnt.at[sid])
        plsc.subcore_barrier()

        pltpu.sync_copy(sh_cnt, all_cnt_v)
        acc = jnp.zeros((L16,), jnp.int32)
        pref = jnp.zeros((L16,), jnp.int32)
        for wp in range(NS):
            row = plsc.load_gather(
                all_cnt_v, [jnp.full((L16,), wp, jnp.int32), lanes])
            pref = jnp.where(jnp.full((L16,), wp, jnp.int32) == sid,
                             acc, pref)
            acc = acc + row
        padded = ((acc + BLK - 1) // BLK) * BLK
        pend = plsc.cumsum(padded)
        myoff = (pend - padded) + pref
        offs = [jnp.sum(jnp.where(lanes == e, myoff, 0))
                for e in range(E)]

        runs = [jnp.int32(0)] * E
        for j in range(TPT // L16):
            a = avs[j]
            tok = base + j * L16 + lanes
            pos = jnp.zeros((L16,), jnp.int32)
            for e in range(E):
                m = a == e
                mi = m.astype(jnp.int32)
                rank = plsc.cumsum(mi) - mi
                pos = jnp.where(m, offs[e] + runs[e] + rank, pos)
                runs[e] = runs[e] + jnp.sum(mi)
            half = j // 8
            slot = (j % 8) * L16 + lanes
            plsc.store_scatter(pos_v, [jnp.full((L16,), half, jnp.int32),
                                       slot], pos)
            plsc.store_scatter(val_v, [jnp.full((L16,), half, jnp.int32),
                                       slot], tok)
        for h in range(TPT // 128):
            pltpu.sync_copy(val_v.at[h], perm_hbm.at[pos_v.at[h]])

        @pl.when(sid == 0)
        def _():
            ends = [jnp.sum(jnp.where(lanes == e, pend, 0))
                    for e in range(E - 1)]
            for v in range(NBPAD // L16):
                bidx = v * L16 + lanes
                bstart = bidx * BLK
                eid = jnp.zeros((L16,), jnp.int32)
                for e in range(E - 1):
                    eid = eid + (bstart >= ends[e]).astype(jnp.int32)
                plsc.store_scatter(bexp_v, [bidx], eid)
            pltpu.sync_copy(bexp_v, bexp_hbm)


@functools.cache
def _route():
    return pl.kernel(
        _route_body,
        out_type=(
            jax.ShapeDtypeStruct((CAP,), jnp.int32),
            jax.ShapeDtypeStruct((NBPAD,), jnp.int32),
        ),
        mesh=_sc_mesh(),
        compiler_params=pltpu.CompilerParams(needs_layout_passes=False),
        scratch_types=[
            pltpu.VMEM((TPT,), jnp.int32),
            pltpu.VMEM((SPT,), jnp.int32),
            pltpu.VMEM((L16,), jnp.int32),
            pltpu.VMEM((NS, L16), jnp.int32),
            pltpu.VMEM((TPT // 128, 128), jnp.int32),
            pltpu.VMEM((TPT // 128, 128), jnp.int32),
            pltpu.VMEM((NBPAD,), jnp.int32),
            pltpu.VMEM_SHARED((NS, L16), jnp.int32),
        ],
    )


@functools.cache
def _gather(nch):
    def body(x_hbm, perm2_hbm, xs_hbm, idx_v, rows_v, sem_i, sem_g, sem_w):
        cid = lax.axis_index("c")
        sid = lax.axis_index("s")
        wid = sid * NC + cid
        lanes = lax.iota(jnp.int32, L16)

        ci = [pltpu.async_copy(perm2_hbm.at[wid * nch + c], idx_v.at[c],
                               sem_i.at[c])
              for c in range(nch)]
        cg = [None] * nch
        cw = [None] * nch
        for c in range(nch):
            ci[c].wait()
            for j in range(CR // L16):
                ii = j * L16 + lanes
                cc = jnp.full((L16,), c, jnp.int32)
                v = plsc.load_gather(idx_v, [cc, ii])
                plsc.store_scatter(idx_v, [cc, ii], jnp.minimum(v, T - 1))
            cg[c] = pltpu.async_copy(x_hbm.at[idx_v.at[c]], rows_v.at[c],
                                     sem_g.at[c])
        for c in range(nch):
            cg[c].wait()
            cw[c] = pltpu.async_copy(rows_v.at[c], xs_hbm.at[wid, c],
                                     sem_w.at[c])
        for c in range(nch):
            cw[c].wait()

    return pl.kernel(
        body,
        out_type=jax.ShapeDtypeStruct((NW, nch, CR, D2), jnp.int32),
        mesh=_sc_mesh(),
        compiler_params=pltpu.CompilerParams(needs_layout_passes=False),
        scratch_types=[
            pltpu.VMEM((nch, CR), jnp.int32),
            pltpu.VMEM((nch, CR, D2), jnp.int32),
            pltpu.SemaphoreType.DMA((nch,)),
            pltpu.SemaphoreType.DMA((nch,)),
            pltpu.SemaphoreType.DMA((nch,)),
        ],
    )


def _mm_body(bexp_ref, xs_ref, ew_ref, eb_ref, ow_ref, ob_ref, out_ref):
    w = xs_ref[...]
    xb = jnp.concatenate(
        [_unpack_lo(w), _unpack_hi(w)], axis=1).astype(jnp.bfloat16)
    h = lax.dot_general(
        xb, ew_ref[0], (((1,), (1,)), ((), ())),
        preferred_element_type=jnp.float32)
    h = (h + eb_ref[0]).astype(jnp.bfloat16)
    o = lax.dot_general(
        h, ow_ref[...], (((1,), (1,)), ((), ())),
        preferred_element_type=jnp.float32)
    o = o + ob_ref[...]
    out_ref[...] = _pack_pair(o[:, :OD2], o[:, OD2:])


def _mm(bexp, xs, expert_W, expert_b, out_W, out_b):
    nblk = xs.shape[0] // BLK
    return pl.pallas_call(
        _mm_body,
        grid_spec=pltpu.PrefetchScalarGridSpec(
            num_scalar_prefetch=1,
            grid=(nblk,),
            in_specs=[
                pl.BlockSpec((BLK, D2), lambda i, b: (i, 0)),
                pl.BlockSpec((1, ED, D), lambda i, b: (b[i], 0, 0)),
                pl.BlockSpec((1, 1, ED), lambda i, b: (b[i], 0, 0)),
                pl.BlockSpec((OD, ED), lambda i, b: (0, 0)),
                pl.BlockSpec((1, OD), lambda i, b: (0, 0)),
            ],
            out_specs=pl.BlockSpec((BLK, OD2), lambda i, b: (i, 0)),
        ),
        out_shape=jax.ShapeDtypeStruct((xs.shape[0], OD2), jnp.int32),
        compiler_params=pltpu.CompilerParams(
            dimension_semantics=("arbitrary",)),
    )(bexp, xs, expert_W, expert_b.reshape(E, 1, ED), out_W,
      out_b.reshape(1, OD))


def _unpack_body(op_ref, out_ref):
    w = op_ref[...]
    out_ref[...] = jnp.concatenate([_unpack_lo(w), _unpack_hi(w)], axis=1)


def _unpack_out(opad):
    blk = 512
    return pl.pallas_call(
        _unpack_body,
        grid=(T // blk,),
        in_specs=[pl.BlockSpec((blk, OD2), lambda i: (i, 0))],
        out_specs=pl.BlockSpec((blk, OD), lambda i: (i, 0)),
        out_shape=jax.ShapeDtypeStruct((T, OD), jnp.float32),
    )(opad)


def _scatter_body(ys0_hbm, ys1_hbm, perm2_hbm, opad_hbm, idx_v, rows_v,
                  sem_i, sem_r, sem_s):
    cid = lax.axis_index("c")
    sid = lax.axis_index("s")
    wid = sid * NC + cid

    ci = [pltpu.async_copy(perm2_hbm.at[wid * CHUNKS + c], idx_v.at[c],
                           sem_i.at[c])
          for c in range(CHUNKS)]

    @pl.when(wid < NS)
    def _():
        cr = [pltpu.async_copy(ys0_hbm.at[wid, c], rows_v.at[c],
                               sem_r.at[c])
              for c in range(CHUNKS)]
        for d in cr:
            d.wait()

    @pl.when(wid >= NS)
    def _():
        cr = [pltpu.async_copy(ys1_hbm.at[wid - NS, c], rows_v.at[c],
                               sem_r.at[c])
              for c in range(CHUNKS)]
        for d in cr:
            d.wait()

    cs = [None] * CHUNKS
    for c in range(CHUNKS):
        ci[c].wait()
        cs[c] = pltpu.async_copy(rows_v.at[c], opad_hbm.at[idx_v.at[c]],
                                 sem_s.at[c])
    for c in range(CHUNKS):
        cs[c].wait()


@functools.cache
def _scatter():
    return pl.kernel(
        _scatter_body,
        out_type=jax.ShapeDtypeStruct((T + 8, OD2), jnp.int32),
        mesh=_sc_mesh(),
        compiler_params=pltpu.CompilerParams(needs_layout_passes=False),
        scratch_types=[
            pltpu.VMEM((CHUNKS, CR), jnp.int32),
            pltpu.VMEM((CHUNKS, CR, OD2), jnp.int32),
            pltpu.SemaphoreType.DMA((CHUNKS,)),
            pltpu.SemaphoreType.DMA((CHUNKS,)),
            pltpu.SemaphoreType.DMA((CHUNKS,)),
        ],
    )


def kernel(x, gate_W, gate_b, expert_W, expert_b, out_W, out_b):
    assert x.shape == (B, S, D)
    assert expert_W.shape == (E, ED, D)
    assert out_W.shape == (OD, ED)

    xf = x.reshape(T, D)
    ewbf = expert_W.astype(jnp.bfloat16)
    owbf = out_W.astype(jnp.bfloat16)
    assign, xp = _gate(xf, gate_W, gate_b)
    perm, bexp = _route_v1()(assign)
    half = CAP // 2
    nbh = NB // 2
    xs0 = _gather(2)(xp, perm[:half].reshape(NW * 2, CR))
    xs1 = _gather(2)(xp, perm[half:].reshape(NW * 2, CR))
    ys0 = _mm(bexp[:nbh], xs0.reshape(half, D2), ewbf, expert_b, owbf,
              out_b)
    ys1 = _mm(bexp[nbh:2 * nbh], xs1.reshape(half, D2), ewbf, expert_b,
              owbf, out_b)
    opad = _scatter()(ys0.reshape(NS, CHUNKS, CR, OD2),
                      ys1.reshape(NS, CHUNKS, CR, OD2),
                      perm.reshape(NW * CHUNKS, CR))
    return _unpack_out(opad).reshape(B, S, OD)


# speedup vs baseline: 2.7010x; 1.1005x over previous
import functools

import jax
import jax.numpy as jnp
from jax import lax
from jax.experimental import pallas as pl
from jax.experimental.pallas import tpu as pltpu
from jax.experimental.pallas import tpu_sc as plsc

B, S, D = 2, 2048, 768
E, ED, OD = 8, 1024, 768
T = B * S

BLK = 128
NB = T // BLK + E
CAP = NB * BLK
NBPAD = 64
L16 = 16
NC, NS = 2, 16
NW = NC * NS
RPW = CAP // NW
CHUNKS = 4
CR = RPW // CHUNKS
TPT = T // NS
SPT = CAP // NS
D2 = D // 2
OD2 = OD // 2


@functools.cache
def _sc_mesh():
    return plsc.VectorSubcoreMesh(
        core_axis_name="c", subcore_axis_name="s",
        num_cores=NC, num_subcores=NS)


def _bf16_bits(v):
    b = lax.bitcast_convert_type(v, jnp.int32)
    return b + 0x7FFF + (lax.shift_right_logical(b, 16) & 1)


def _pack_pair(lo, hi):
    return (_bf16_bits(hi) & jnp.int32(-65536)) | lax.shift_right_logical(
        _bf16_bits(lo), 16)


def _unpack_lo(w):
    return lax.bitcast_convert_type(lax.shift_left(w, 16), jnp.float32)


def _unpack_hi(w):
    return lax.bitcast_convert_type(w & jnp.int32(-65536), jnp.float32)


def _gate_body(x_ref, gw_ref, gb_ref, out_ref, xp_ref):
    xv = x_ref[...]
    logits = lax.dot_general(
        xv, gw_ref[...], (((1,), (1,)), ((), ())),
        preferred_element_type=jnp.float32)
    logits = logits + gb_ref[...]
    out_ref[...] = jnp.argmax(logits, axis=-1).astype(jnp.int32)
    xp_ref[...] = _pack_pair(xv[:, :D2], xv[:, D2:])


def _gate(xf, gate_W, gate_b):
    blk = 512
    return pl.pallas_call(
        _gate_body,
        grid=(T // blk,),
        in_specs=[
            pl.BlockSpec((blk, D), lambda i: (i, 0)),
            pl.BlockSpec((E, D), lambda i: (0, 0)),
            pl.BlockSpec((1, E), lambda i: (0, 0)),
        ],
        out_specs=[
            pl.BlockSpec((blk,), lambda i: (i,)),
            pl.BlockSpec((blk, D2), lambda i: (i, 0)),
        ],
        out_shape=[
            jax.ShapeDtypeStruct((T,), jnp.int32),
            jax.ShapeDtypeStruct((T, D2), jnp.int32),
        ],
    )(xf, gate_W, gate_b.reshape(1, E))


def _route_body_v1(assign_hbm, perm_hbm, bexp_hbm,
                   assign_v, perm_v, bexp_v):
    cid = lax.axis_index("c")
    sid = lax.axis_index("s")

    @pl.when(jnp.logical_and(cid == 0, sid == 0))
    def _():
        pltpu.sync_copy(assign_hbm, assign_v)
        lanes = lax.iota(jnp.int32, L16)

        def init_body(j, carry):
            idx = j * L16 + lanes
            plsc.store_scatter(perm_v, [idx], jnp.full((L16,), T, jnp.int32))
            return carry

        lax.fori_loop(0, CAP // L16, init_body, jnp.int32(0))

        ends = []
        start = jnp.int32(0)
        for e in range(E):
            def scan_body(j, cnt, e=e, start=start):
                idx = j * L16 + lanes
                a = plsc.load_gather(assign_v, [idx])
                m = a == e
                mi = m.astype(jnp.int32)
                rank = plsc.cumsum(mi) - mi
                pos = start + cnt + rank
                plsc.store_scatter(perm_v, [pos], idx, mask=m)
                return cnt + jnp.sum(mi)

            cnt = lax.fori_loop(0, T // L16, scan_body, jnp.int32(0))
            nblk = (cnt + BLK - 1) // BLK
            start = start + nblk * BLK
            ends.append(start)

        for v in range(NBPAD // L16):
            bidx = v * L16 + lanes
            bstart = bidx * BLK
            eid = jnp.zeros((L16,), jnp.int32)
            for e in range(E - 1):
                eid = eid + (bstart >= ends[e]).astype(jnp.int32)
            plsc.store_scatter(bexp_v, [bidx], eid)

        pltpu.sync_copy(perm_v, perm_hbm)
        pltpu.sync_copy(bexp_v, bexp_hbm)


@functools.cache
def _route_v1():
    return pl.kernel(
        _route_body_v1,
        out_type=(
            jax.ShapeDtypeStruct((CAP,), jnp.int32),
            jax.ShapeDtypeStruct((NBPAD,), jnp.int32),
        ),
        mesh=_sc_mesh(),
        compiler_params=pltpu.CompilerParams(needs_layout_passes=False),
        scratch_types=[
            pltpu.VMEM((T,), jnp.int32),
            pltpu.VMEM((CAP,), jnp.int32),
            pltpu.VMEM((NBPAD,), jnp.int32),
        ],
    )


def _route_body_v2(assign_hbm, perm_hbm, bexp_hbm,
                   assign_v, toks_v, cnt_v, all_cnt_v, bexp_v, sh_cnt):
    cid = lax.axis_index("c")
    sid = lax.axis_index("s")

    @pl.when(cid == 0)
    def _():
        lanes = lax.iota(jnp.int32, L16)
        fill = jnp.full((L16,), T, jnp.int32)

        @pl.when(sid < E)
        def _():
            pltpu.sync_copy(assign_hbm, assign_v)
            def fill_body(j, carry):
                plsc.store_scatter(toks_v, [j * L16 + lanes], fill)
                return carry

            lax.fori_loop(0, (T + BLK) // L16, fill_body, jnp.int32(0))

            def scan_body(j, cnt):
                idx = j * L16 + lanes
                a = plsc.load_gather(assign_v, [idx])
                m = a == sid
                mi = m.astype(jnp.int32)
                rank = plsc.cumsum(mi) - mi
                plsc.store_scatter(toks_v, [cnt + rank], idx, mask=m)
                return cnt + jnp.sum(mi)

            cnt = lax.fori_loop(0, T // L16, scan_body, jnp.int32(0))
            plsc.store_scatter(cnt_v, [lanes],
                              jnp.full((L16,), cnt, jnp.int32))
            pltpu.sync_copy(cnt_v, sh_cnt.at[sid])

        plsc.subcore_barrier()
        pltpu.sync_copy(sh_cnt, all_cnt_v)
        counts = [
            jnp.sum(plsc.load_gather(
                all_cnt_v, [jnp.full((L16,), e, jnp.int32), lanes])
                * (lanes == 0).astype(jnp.int32)) for e in range(E)]
        nblks = [(c + BLK - 1) // BLK for c in counts]
        starts = []
        cur = jnp.int32(0)
        for e in range(E):
            starts.append(cur)
            cur = cur + nblks[e] * BLK

        @pl.when(sid < E)
        def _():
            def out_body(k, carry):
                src_off = pl.multiple_of(k * BLK, BLK)
                dst_off = pl.multiple_of(_pick(starts, sid) + k * BLK, BLK)
                pltpu.sync_copy(
                    toks_v.at[pl.ds(src_off, BLK)],
                    perm_hbm.at[pl.ds(dst_off, BLK)])
                return carry

            lax.fori_loop(0, _pick(nblks, sid), out_body, jnp.int32(0))

        @pl.when(sid == E)
        def _():
            def fill_body(j, carry):
                plsc.store_scatter(toks_v, [j * L16 + lanes], fill)
                return carry

            lax.fori_loop(0, BLK // L16, fill_body, jnp.int32(0))

            def tail_body(k, carry):
                dst_off = pl.multiple_of(cur + k * BLK, BLK)
                pltpu.sync_copy(
                    toks_v.at[pl.ds(0, BLK)],
                    perm_hbm.at[pl.ds(dst_off, BLK)])
                return carry

            lax.fori_loop(0, (CAP - cur) // BLK, tail_body, jnp.int32(0))

            ends = []
            run = jnp.int32(0)
            for e in range(E):
                run = run + nblks[e] * BLK
                ends.append(run)
            for v in range(NBPAD // L16):
                bidx = v * L16 + lanes
                bstart = bidx * BLK
                eid = jnp.zeros((L16,), jnp.int32)
                for e in range(E - 1):
                    eid = eid + (bstart >= ends[e]).astype(jnp.int32)
                plsc.store_scatter(bexp_v, [bidx], eid)
            pltpu.sync_copy(bexp_v, bexp_hbm)


def _pick(vals, sel):
    out = vals[0]
    for i in range(1, len(vals)):
        out = jnp.where(sel == i, vals[i], out)
    return out


@functools.cache
def _route_v2():
    return pl.kernel(
        _route_body_v2,
        out_type=(
            jax.ShapeDtypeStruct((CAP,), jnp.int32),
            jax.ShapeDtypeStruct((NBPAD,), jnp.int32),
        ),
        mesh=_sc_mesh(),
        compiler_params=pltpu.CompilerParams(needs_layout_passes=False),
        scratch_types=[
            pltpu.VMEM((T,), jnp.int32),
            pltpu.VMEM((T + BLK,), jnp.int32),
            pltpu.VMEM((L16,), jnp.int32),
            pltpu.VMEM((E, L16), jnp.int32),
            pltpu.VMEM((NBPAD,), jnp.int32),
            pltpu.VMEM_SHARED((E, L16), jnp.int32),
        ],
    )


def _route_body(assign_hbm, perm_hbm, bexp_hbm,
                assign_v, fill_v, cnt_v, all_cnt_v, pos_v, val_v, bexp_v,
                sh_cnt):
    cid = lax.axis_index("c")
    sid = lax.axis_index("s")

    @pl.when(cid == 0)
    def _():
        lanes = lax.iota(jnp.int32, L16)
        base = sid * TPT
        pltpu.sync_copy(assign_hbm.at[pl.ds(base, TPT)], assign_v)

        fill = jnp.full((L16,), T, jnp.int32)
        for j in range(SPT // L16):
            plsc.store_scatter(fill_v, [j * L16 + lanes], fill)
        pltpu.sync_copy(fill_v, perm_hbm.at[pl.ds(sid * SPT, SPT)])

        avs = [plsc.load_gather(assign_v, [j * L16 + lanes])
               for j in range(TPT // L16)]
        cnts = []
        for e in range(E):
            c = jnp.int32(0)
            for a in avs:
                c = c + jnp.sum((a == e).astype(jnp.int32))
            cnts.append(c)
        cnt_vec = jnp.zeros((L16,), jnp.int32)
        for e in range(E):
            cnt_vec = jnp.where(lanes == e, cnts[e], cnt_vec)
        plsc.store_scatter(cnt_v, [lanes], cnt_vec)
        pltpu.sync_copy(cnt_v, sh_cnt.at[sid])
        plsc.subcore_barrier()

        pltpu.sync_copy(sh_cnt, all_cnt_v)
        acc = jnp.zeros((L16,), jnp.int32)
        pref = jnp.zeros((L16,), jnp.int32)
        for wp in range(NS):
            row = plsc.load_gather(
                all_cnt_v, [jnp.full((L16,), wp, jnp.int32), lanes])
            pref = jnp.where(jnp.full((L16,), wp, jnp.int32) == sid,
                             acc, pref)
            acc = acc + row
        padded = ((acc + BLK - 1) // BLK) * BLK
        pend = plsc.cumsum(padded)
        myoff = (pend - padded) + pref
        offs = [jnp.sum(jnp.where(lanes == e, myoff, 0))
                for e in range(E)]

        runs = [jnp.int32(0)] * E
        for j in range(TPT // L16):
            a = avs[j]
            tok = base + j * L16 + lanes
            pos = jnp.zeros((L16,), jnp.int32)
            for e in range(E):
                m = a == e
                mi = m.astype(jnp.int32)
                rank = plsc.cumsum(mi) - mi
                pos = jnp.where(m, offs[e] + runs[e] + rank, pos)
                runs[e] = runs[e] + jnp.sum(mi)
            half = j // 8
            slot = (j % 8) * L16 + lanes
            plsc.store_scatter(pos_v, [jnp.full((L16,), half, jnp.int32),
                                       slot], pos)
            plsc.store_scatter(val_v, [jnp.full((L16,), half, jnp.int32),
                                       slot], tok)
        for h in range(TPT // 128):
            pltpu.sync_copy(val_v.at[h], perm_hbm.at[pos_v.at[h]])

        @pl.when(sid == 0)
        def _():
            ends = [jnp.sum(jnp.where(lanes == e, pend, 0))
                    for e in range(E - 1)]
            for v in range(NBPAD // L16):
                bidx = v * L16 + lanes
                bstart = bidx * BLK
                eid = jnp.zeros((L16,), jnp.int32)
                for e in range(E - 1):
                    eid = eid + (bstart >= ends[e]).astype(jnp.int32)
                plsc.store_scatter(bexp_v, [bidx], eid)
            pltpu.sync_copy(bexp_v, bexp_hbm)


@functools.cache
def _route():
    return pl.kernel(
        _route_body,
        out_type=(
            jax.ShapeDtypeStruct((CAP,), jnp.int32),
            jax.ShapeDtypeStruct((NBPAD,), jnp.int32),
        ),
        mesh=_sc_mesh(),
        compiler_params=pltpu.CompilerParams(needs_layout_passes=False),
        scratch_types=[
            pltpu.VMEM((TPT,), jnp.int32),
            pltpu.VMEM((SPT,), jnp.int32),
            pltpu.VMEM((L16,), jnp.int32),
            pltpu.VMEM((NS, L16), jnp.int32),
            pltpu.VMEM((TPT // 128, 128), jnp.int32),
            pltpu.VMEM((TPT // 128, 128), jnp.int32),
            pltpu.VMEM((NBPAD,), jnp.int32),
            pltpu.VMEM_SHARED((NS, L16), jnp.int32),
        ],
    )


@functools.cache
def _gather(nch):
    def body(x_hbm, perm2_hbm, xs_hbm, idx_v, rows_v, sem_i, sem_g, sem_w):
        cid = lax.axis_index("c")
        sid = lax.axis_index("s")
        wid = sid * NC + cid
        lanes = lax.iota(jnp.int32, L16)

        ci = [pltpu.async_copy(perm2_hbm.at[wid * nch + c], idx_v.at[c],
                               sem_i.at[c])
              for c in range(nch)]
        cg = [None] * nch
        cw = [None] * nch
        for c in range(nch):
            ci[c].wait()
            for j in range(CR // L16):
                ii = j * L16 + lanes
                cc = jnp.full((L16,), c, jnp.int32)
                v = plsc.load_gather(idx_v, [cc, ii])
                plsc.store_scatter(idx_v, [cc, ii], jnp.minimum(v, T - 1))
            cg[c] = pltpu.async_copy(x_hbm.at[idx_v.at[c]], rows_v.at[c],
                                     sem_g.at[c])
        for c in range(nch):
            cg[c].wait()
            cw[c] = pltpu.async_copy(rows_v.at[c], xs_hbm.at[wid, c],
                                     sem_w.at[c])
        for c in range(nch):
            cw[c].wait()

    return pl.kernel(
        body,
        out_type=jax.ShapeDtypeStruct((NW, nch, CR, D2), jnp.int32),
        mesh=_sc_mesh(),
        compiler_params=pltpu.CompilerParams(needs_layout_passes=False),
        scratch_types=[
            pltpu.VMEM((nch, CR), jnp.int32),
            pltpu.VMEM((nch, CR, D2), jnp.int32),
            pltpu.SemaphoreType.DMA((nch,)),
            pltpu.SemaphoreType.DMA((nch,)),
            pltpu.SemaphoreType.DMA((nch,)),
        ],
    )


def _mm_body(bexp_ref, xs_ref, ew_ref, eb_ref, ow_ref, ob_ref, out_ref):
    w = xs_ref[...]
    xb = jnp.concatenate(
        [_unpack_lo(w), _unpack_hi(w)], axis=1).astype(jnp.bfloat16)
    h = lax.dot_general(
        xb, ew_ref[0], (((1,), (1,)), ((), ())),
        preferred_element_type=jnp.float32)
    h = (h + eb_ref[0]).astype(jnp.bfloat16)
    o = lax.dot_general(
        h, ow_ref[...], (((1,), (1,)), ((), ())),
        preferred_element_type=jnp.float32)
    o = o + ob_ref[...]
    out_ref[...] = _pack_pair(o[:, :OD2], o[:, OD2:])


def _mm(bexp, xs, expert_W, expert_b, out_W, out_b):
    nblk = xs.shape[0] // BLK
    return pl.pallas_call(
        _mm_body,
        grid_spec=pltpu.PrefetchScalarGridSpec(
            num_scalar_prefetch=1,
            grid=(nblk,),
            in_specs=[
                pl.BlockSpec((BLK, D2), lambda i, b: (i, 0)),
                pl.BlockSpec((1, ED, D), lambda i, b: (b[i], 0, 0)),
                pl.BlockSpec((1, 1, ED), lambda i, b: (b[i], 0, 0)),
                pl.BlockSpec((OD, ED), lambda i, b: (0, 0)),
                pl.BlockSpec((1, OD), lambda i, b: (0, 0)),
            ],
            out_specs=pl.BlockSpec((BLK, OD2), lambda i, b: (i, 0)),
        ),
        out_shape=jax.ShapeDtypeStruct((xs.shape[0], OD2), jnp.int32),
        compiler_params=pltpu.CompilerParams(
            dimension_semantics=("arbitrary",)),
    )(bexp, xs, expert_W, expert_b.reshape(E, 1, ED), out_W,
      out_b.reshape(1, OD))


def _unpack_body(op_ref, out_ref):
    w = op_ref[...]
    out_ref[...] = jnp.concatenate([_unpack_lo(w), _unpack_hi(w)], axis=1)


def _unpack_out(opad):
    blk = 512
    return pl.pallas_call(
        _unpack_body,
        grid=(T // blk,),
        in_specs=[pl.BlockSpec((blk, OD2), lambda i: (i, 0))],
        out_specs=pl.BlockSpec((blk, OD), lambda i: (i, 0)),
        out_shape=jax.ShapeDtypeStruct((T, OD), jnp.float32),
    )(opad)


def _scatter_body(ys0_hbm, ys1_hbm, perm2_hbm, opad_hbm, idx_v, rows_v,
                  sem_i, sem_r, sem_s):
    cid = lax.axis_index("c")
    sid = lax.axis_index("s")
    wid = sid * NC + cid

    ci = [pltpu.async_copy(perm2_hbm.at[wid * CHUNKS + c], idx_v.at[c],
                           sem_i.at[c])
          for c in range(CHUNKS)]

    @pl.when(wid < NS)
    def _():
        cr = [pltpu.async_copy(ys0_hbm.at[wid, c], rows_v.at[c],
                               sem_r.at[c])
              for c in range(CHUNKS)]
        for d in cr:
            d.wait()

    @pl.when(wid >= NS)
    def _():
        cr = [pltpu.async_copy(ys1_hbm.at[wid - NS, c], rows_v.at[c],
                               sem_r.at[c])
              for c in range(CHUNKS)]
        for d in cr:
            d.wait()

    cs = [None] * CHUNKS
    for c in range(CHUNKS):
        ci[c].wait()
        cs[c] = pltpu.async_copy(rows_v.at[c], opad_hbm.at[idx_v.at[c]],
                                 sem_s.at[c])
    for c in range(CHUNKS):
        cs[c].wait()


@functools.cache
def _scatter():
    return pl.kernel(
        _scatter_body,
        out_type=jax.ShapeDtypeStruct((T + 8, OD2), jnp.int32),
        mesh=_sc_mesh(),
        compiler_params=pltpu.CompilerParams(needs_layout_passes=False),
        scratch_types=[
            pltpu.VMEM((CHUNKS, CR), jnp.int32),
            pltpu.VMEM((CHUNKS, CR, OD2), jnp.int32),
            pltpu.SemaphoreType.DMA((CHUNKS,)),
            pltpu.SemaphoreType.DMA((CHUNKS,)),
            pltpu.SemaphoreType.DMA((CHUNKS,)),
        ],
    )


def kernel(x, gate_W, gate_b, expert_W, expert_b, out_W, out_b):
    assert x.shape == (B, S, D)
    assert expert_W.shape == (E, ED, D)
    assert out_W.shape == (OD, ED)

    xf = x.reshape(T, D)
    ewbf = expert_W.astype(jnp.bfloat16)
    owbf = out_W.astype(jnp.bfloat16)
    assign, xp = _gate(xf, gate_W, gate_b)
    perm, bexp = _route_v2()(assign)
    half = CAP // 2
    nbh = NB // 2
    xs0 = _gather(2)(xp, perm[:half].reshape(NW * 2, CR))
    xs1 = _gather(2)(xp, perm[half:].reshape(NW * 2, CR))
    ys0 = _mm(bexp[:nbh], xs0.reshape(half, D2), ewbf, expert_b, owbf,
              out_b)
    ys1 = _mm(bexp[nbh:2 * nbh], xs1.reshape(half, D2), ewbf, expert_b,
              owbf, out_b)
    opad = _scatter()(ys0.reshape(NS, CHUNKS, CR, OD2),
                      ys1.reshape(NS, CHUNKS, CR, OD2),
                      perm.reshape(NW * CHUNKS, CR))
    return _unpack_out(opad).reshape(B, S, OD)


# speedup vs baseline: 2.8042x; 1.0382x over previous
import functools

import jax
import jax.numpy as jnp
from jax import lax
from jax.experimental import pallas as pl
from jax.experimental.pallas import tpu as pltpu
from jax.experimental.pallas import tpu_sc as plsc

B, S, D = 2, 2048, 768
E, ED, OD = 8, 1024, 768
T = B * S

BLK = 128
NB = T // BLK + E
CAP = NB * BLK
NBPAD = 64
L16 = 16
NC, NS = 2, 16
NW = NC * NS
RPW = CAP // NW
CHUNKS = 4
CR = RPW // CHUNKS
TPT = T // NS
SPT = CAP // NS
D2 = D // 2
OD2 = OD // 2


@functools.cache
def _sc_mesh():
    return plsc.VectorSubcoreMesh(
        core_axis_name="c", subcore_axis_name="s",
        num_cores=NC, num_subcores=NS)


def _bf16_bits(v):
    b = lax.bitcast_convert_type(v, jnp.int32)
    return b + 0x7FFF + (lax.shift_right_logical(b, 16) & 1)


def _pack_pair(lo, hi):
    return (_bf16_bits(hi) & jnp.int32(-65536)) | lax.shift_right_logical(
        _bf16_bits(lo), 16)


def _unpack_lo(w):
    return lax.bitcast_convert_type(lax.shift_left(w, 16), jnp.float32)


def _unpack_hi(w):
    return lax.bitcast_convert_type(w & jnp.int32(-65536), jnp.float32)


def _gate_body(x_ref, gw_ref, gb_ref, out_ref, xp_ref):
    xv = x_ref[...]
    logits = lax.dot_general(
        xv, gw_ref[...], (((1,), (1,)), ((), ())),
        preferred_element_type=jnp.float32)
    logits = logits + gb_ref[...]
    out_ref[...] = jnp.argmax(logits, axis=-1).astype(jnp.int32)
    xp_ref[...] = _pack_pair(xv[:, :D2], xv[:, D2:])


def _gate(xf, gate_W, gate_b):
    blk = 512
    return pl.pallas_call(
        _gate_body,
        grid=(T // blk,),
        in_specs=[
            pl.BlockSpec((blk, D), lambda i: (i, 0)),
            pl.BlockSpec((E, D), lambda i: (0, 0)),
            pl.BlockSpec((1, E), lambda i: (0, 0)),
        ],
        out_specs=[
            pl.BlockSpec((blk,), lambda i: (i,)),
            pl.BlockSpec((blk, D2), lambda i: (i, 0)),
        ],
        out_shape=[
            jax.ShapeDtypeStruct((T,), jnp.int32),
            jax.ShapeDtypeStruct((T, D2), jnp.int32),
        ],
    )(xf, gate_W, gate_b.reshape(1, E))


def _route_body_v1(assign_hbm, perm_hbm, bexp_hbm,
                   assign_v, perm_v, bexp_v):
    cid = lax.axis_index("c")
    sid = lax.axis_index("s")

    @pl.when(jnp.logical_and(cid == 0, sid == 0))
    def _():
        pltpu.sync_copy(assign_hbm, assign_v)
        lanes = lax.iota(jnp.int32, L16)

        def init_body(j, carry):
            idx = j * L16 + lanes
            plsc.store_scatter(perm_v, [idx], jnp.full((L16,), T, jnp.int32))
            return carry

        lax.fori_loop(0, CAP // L16, init_body, jnp.int32(0))

        ends = []
        start = jnp.int32(0)
        for e in range(E):
            def scan_body(j, cnt, e=e, start=start):
                idx = j * L16 + lanes
                a = plsc.load_gather(assign_v, [idx])
                m = a == e
                mi = m.astype(jnp.int32)
                rank = plsc.cumsum(mi) - mi
                pos = start + cnt + rank
                plsc.store_scatter(perm_v, [pos], idx, mask=m)
                return cnt + jnp.sum(mi)

            cnt = lax.fori_loop(0, T // L16, scan_body, jnp.int32(0))
            nblk = (cnt + BLK - 1) // BLK
            start = start + nblk * BLK
            ends.append(start)

        for v in range(NBPAD // L16):
            bidx = v * L16 + lanes
            bstart = bidx * BLK
            eid = jnp.zeros((L16,), jnp.int32)
            for e in range(E - 1):
                eid = eid + (bstart >= ends[e]).astype(jnp.int32)
            plsc.store_scatter(bexp_v, [bidx], eid)

        pltpu.sync_copy(perm_v, perm_hbm)
        pltpu.sync_copy(bexp_v, bexp_hbm)


@functools.cache
def _route_v1():
    return pl.kernel(
        _route_body_v1,
        out_type=(
            jax.ShapeDtypeStruct((CAP,), jnp.int32),
            jax.ShapeDtypeStruct((NBPAD,), jnp.int32),
        ),
        mesh=_sc_mesh(),
        compiler_params=pltpu.CompilerParams(needs_layout_passes=False),
        scratch_types=[
            pltpu.VMEM((T,), jnp.int32),
            pltpu.VMEM((CAP,), jnp.int32),
            pltpu.VMEM((NBPAD,), jnp.int32),
        ],
    )


def _route_body_v2(assign_hbm, perm_hbm, bexp_hbm,
                   assign_v, toks_v, cnt_v, all_cnt_v, bexp_v, sh_cnt):
    cid = lax.axis_index("c")
    sid = lax.axis_index("s")

    @pl.when(cid == 0)
    def _():
        lanes = lax.iota(jnp.int32, L16)
        fill = jnp.full((L16,), T, jnp.int32)

        @pl.when(sid < E)
        def _():
            pltpu.sync_copy(assign_hbm, assign_v)
            def fill_body(j, carry):
                plsc.store_scatter(toks_v, [j * L16 + lanes], fill)
                return carry

            lax.fori_loop(0, (T + BLK) // L16, fill_body, jnp.int32(0))

            def scan_body(j, cnt):
                idx = j * L16 + lanes
                a = plsc.load_gather(assign_v, [idx])
                m = a == sid
                mi = m.astype(jnp.int32)
                rank = plsc.cumsum(mi) - mi
                plsc.store_scatter(toks_v, [cnt + rank], idx, mask=m)
                return cnt + jnp.sum(mi)

            cnt = lax.fori_loop(0, T // L16, scan_body, jnp.int32(0))
            plsc.store_scatter(cnt_v, [lanes],
                              jnp.full((L16,), cnt, jnp.int32))
            pltpu.sync_copy(cnt_v, sh_cnt.at[sid])

        plsc.subcore_barrier()
        pltpu.sync_copy(sh_cnt, all_cnt_v)
        counts = [
            jnp.sum(plsc.load_gather(
                all_cnt_v, [jnp.full((L16,), e, jnp.int32), lanes])
                * (lanes == 0).astype(jnp.int32)) for e in range(E)]
        nblks = [(c + BLK - 1) // BLK for c in counts]
        starts = []
        cur = jnp.int32(0)
        for e in range(E):
            starts.append(cur)
            cur = cur + nblks[e] * BLK

        @pl.when(sid < E)
        def _():
            def out_body(k, carry):
                src_off = pl.multiple_of(k * BLK, BLK)
                dst_off = pl.multiple_of(_pick(starts, sid) + k * BLK, BLK)
                pltpu.sync_copy(
                    toks_v.at[pl.ds(src_off, BLK)],
                    perm_hbm.at[pl.ds(dst_off, BLK)])
                return carry

            lax.fori_loop(0, _pick(nblks, sid), out_body, jnp.int32(0))

        @pl.when(sid == E)
        def _():
            def fill_body(j, carry):
                plsc.store_scatter(toks_v, [j * L16 + lanes], fill)
                return carry

            lax.fori_loop(0, BLK // L16, fill_body, jnp.int32(0))

            def tail_body(k, carry):
                dst_off = pl.multiple_of(cur + k * BLK, BLK)
                pltpu.sync_copy(
                    toks_v.at[pl.ds(0, BLK)],
                    perm_hbm.at[pl.ds(dst_off, BLK)])
                return carry

            lax.fori_loop(0, (CAP - cur) // BLK, tail_body, jnp.int32(0))

            ends = []
            run = jnp.int32(0)
            for e in range(E):
                run = run + nblks[e] * BLK
                ends.append(run)
            for v in range(NBPAD // L16):
                bidx = v * L16 + lanes
                bstart = bidx * BLK
                eid = jnp.zeros((L16,), jnp.int32)
                for e in range(E - 1):
                    eid = eid + (bstart >= ends[e]).astype(jnp.int32)
                plsc.store_scatter(bexp_v, [bidx], eid)
            pltpu.sync_copy(bexp_v, bexp_hbm)


def _pick(vals, sel):
    out = vals[0]
    for i in range(1, len(vals)):
        out = jnp.where(sel == i, vals[i], out)
    return out


@functools.cache
def _route_v2():
    return pl.kernel(
        _route_body_v2,
        out_type=(
            jax.ShapeDtypeStruct((CAP,), jnp.int32),
            jax.ShapeDtypeStruct((NBPAD,), jnp.int32),
        ),
        mesh=_sc_mesh(),
        compiler_params=pltpu.CompilerParams(needs_layout_passes=False),
        scratch_types=[
            pltpu.VMEM((T,), jnp.int32),
            pltpu.VMEM((T + BLK,), jnp.int32),
            pltpu.VMEM((L16,), jnp.int32),
            pltpu.VMEM((E, L16), jnp.int32),
            pltpu.VMEM((NBPAD,), jnp.int32),
            pltpu.VMEM_SHARED((E, L16), jnp.int32),
        ],
    )


def _route_body(assign_hbm, perm_hbm, bexp_hbm,
                assign_v, fill_v, cnt_v, all_cnt_v, pos_v, val_v, bexp_v,
                sh_cnt):
    cid = lax.axis_index("c")
    sid = lax.axis_index("s")

    @pl.when(cid == 0)
    def _():
        lanes = lax.iota(jnp.int32, L16)
        base = sid * TPT
        pltpu.sync_copy(assign_hbm.at[pl.ds(base, TPT)], assign_v)

        fill = jnp.full((L16,), T, jnp.int32)
        for j in range(SPT // L16):
            plsc.store_scatter(fill_v, [j * L16 + lanes], fill)
        pltpu.sync_copy(fill_v, perm_hbm.at[pl.ds(sid * SPT, SPT)])

        avs = [plsc.load_gather(assign_v, [j * L16 + lanes])
               for j in range(TPT // L16)]
        cnts = []
        for e in range(E):
            c = jnp.int32(0)
            for a in avs:
                c = c + jnp.sum((a == e).astype(jnp.int32))
            cnts.append(c)
        cnt_vec = jnp.zeros((L16,), jnp.int32)
        for e in range(E):
            cnt_vec = jnp.where(lanes == e, cnts[e], cnt_vec)
        plsc.store_scatter(cnt_v, [lanes], cnt_vec)
        pltpu.sync_copy(cnt_v, sh_cnt.at[sid])
        plsc.subcore_barrier()

        pltpu.sync_copy(sh_cnt, all_cnt_v)
        acc = jnp.zeros((L16,), jnp.int32)
        pref = jnp.zeros((L16,), jnp.int32)
        for wp in range(NS):
            row = plsc.load_gather(
                all_cnt_v, [jnp.full((L16,), wp, jnp.int32), lanes])
            pref = jnp.where(jnp.full((L16,), wp, jnp.int32) == sid,
                             acc, pref)
            acc = acc + row
        padded = ((acc + BLK - 1) // BLK) * BLK
        pend = plsc.cumsum(padded)
        myoff = (pend - padded) + pref
        offs = [jnp.sum(jnp.where(lanes == e, myoff, 0))
                for e in range(E)]

        runs = [jnp.int32(0)] * E
        for j in range(TPT // L16):
            a = avs[j]
            tok = base + j * L16 + lanes
            pos = jnp.zeros((L16,), jnp.int32)
            for e in range(E):
                m = a == e
                mi = m.astype(jnp.int32)
                rank = plsc.cumsum(mi) - mi
                pos = jnp.where(m, offs[e] + runs[e] + rank, pos)
                runs[e] = runs[e] + jnp.sum(mi)
            half = j // 8
            slot = (j % 8) * L16 + lanes
            plsc.store_scatter(pos_v, [jnp.full((L16,), half, jnp.int32),
                                       slot], pos)
            plsc.store_scatter(val_v, [jnp.full((L16,), half, jnp.int32),
                                       slot], tok)
        for h in range(TPT // 128):
            pltpu.sync_copy(val_v.at[h], perm_hbm.at[pos_v.at[h]])

        @pl.when(sid == 0)
        def _():
            ends = [jnp.sum(jnp.where(lanes == e, pend, 0))
                    for e in range(E - 1)]
            for v in range(NBPAD // L16):
                bidx = v * L16 + lanes
                bstart = bidx * BLK
                eid = jnp.zeros((L16,), jnp.int32)
                for e in range(E - 1):
                    eid = eid + (bstart >= ends[e]).astype(jnp.int32)
                plsc.store_scatter(bexp_v, [bidx], eid)
            pltpu.sync_copy(bexp_v, bexp_hbm)


@functools.cache
def _route():
    return pl.kernel(
        _route_body,
        out_type=(
            jax.ShapeDtypeStruct((CAP,), jnp.int32),
            jax.ShapeDtypeStruct((NBPAD,), jnp.int32),
        ),
        mesh=_sc_mesh(),
        compiler_params=pltpu.CompilerParams(needs_layout_passes=False),
        scratch_types=[
            pltpu.VMEM((TPT,), jnp.int32),
            pltpu.VMEM((SPT,), jnp.int32),
            pltpu.VMEM((L16,), jnp.int32),
            pltpu.VMEM((NS, L16), jnp.int32),
            pltpu.VMEM((TPT // 128, 128), jnp.int32),
            pltpu.VMEM((TPT // 128, 128), jnp.int32),
            pltpu.VMEM((NBPAD,), jnp.int32),
            pltpu.VMEM_SHARED((NS, L16), jnp.int32),
        ],
    )


@functools.cache
def _gather(nch):
    def body(x_hbm, perm2_hbm, xs_hbm, idx_v, rows_v, sem_i, sem_g, sem_w):
        cid = lax.axis_index("c")
        sid = lax.axis_index("s")
        wid = sid * NC + cid
        lanes = lax.iota(jnp.int32, L16)

        ci = [pltpu.async_copy(perm2_hbm.at[wid * nch + c], idx_v.at[c],
                               sem_i.at[c])
              for c in range(nch)]
        cg = [None] * nch
        cw = [None] * nch
        for c in range(nch):
            ci[c].wait()
            for j in range(CR // L16):
                ii = j * L16 + lanes
                cc = jnp.full((L16,), c, jnp.int32)
                v = plsc.load_gather(idx_v, [cc, ii])
                plsc.store_scatter(idx_v, [cc, ii], jnp.minimum(v, T - 1))
            cg[c] = pltpu.async_copy(x_hbm.at[idx_v.at[c]], rows_v.at[c],
                                     sem_g.at[c])
        for c in range(nch):
            cg[c].wait()
            cw[c] = pltpu.async_copy(rows_v.at[c], xs_hbm.at[wid, c],
                                     sem_w.at[c])
        for c in range(nch):
            cw[c].wait()

    return pl.kernel(
        body,
        out_type=jax.ShapeDtypeStruct((NW, nch, CR, D2), jnp.int32),
        mesh=_sc_mesh(),
        compiler_params=pltpu.CompilerParams(needs_layout_passes=False),
        scratch_types=[
            pltpu.VMEM((nch, CR), jnp.int32),
            pltpu.VMEM((nch, CR, D2), jnp.int32),
            pltpu.SemaphoreType.DMA((nch,)),
            pltpu.SemaphoreType.DMA((nch,)),
            pltpu.SemaphoreType.DMA((nch,)),
        ],
    )


def _mm_body(bexp_ref, xs_ref, ew_ref, eb_ref, ow_ref, ob_ref, out_ref,
             ewb_scr, owb_scr):
    i = pl.program_id(0)

    @pl.when(i == 0)
    def _():
        owb_scr[...] = ow_ref[...].astype(jnp.bfloat16)

    prev = bexp_ref[jnp.maximum(i - 1, 0)]

    @pl.when(jnp.logical_or(i == 0, bexp_ref[i] != prev))
    def _():
        ewb_scr[...] = ew_ref[0].astype(jnp.bfloat16)

    w = xs_ref[...]
    xb = jnp.concatenate(
        [_unpack_lo(w), _unpack_hi(w)], axis=1).astype(jnp.bfloat16)
    h = lax.dot_general(
        xb, ewb_scr[...], (((1,), (1,)), ((), ())),
        preferred_element_type=jnp.float32)
    h = (h + eb_ref[0]).astype(jnp.bfloat16)
    o = lax.dot_general(
        h, owb_scr[...], (((1,), (1,)), ((), ())),
        preferred_element_type=jnp.float32)
    o = o + ob_ref[...]
    out_ref[...] = _pack_pair(o[:, :OD2], o[:, OD2:])


def _mm(bexp, xs, expert_W, expert_b, out_W, out_b):
    nblk = xs.shape[0] // BLK
    return pl.pallas_call(
        _mm_body,
        grid_spec=pltpu.PrefetchScalarGridSpec(
            num_scalar_prefetch=1,
            grid=(nblk,),
            in_specs=[
                pl.BlockSpec((BLK, D2), lambda i, b: (i, 0)),
                pl.BlockSpec((1, ED, D), lambda i, b: (b[i], 0, 0)),
                pl.BlockSpec((1, 1, ED), lambda i, b: (b[i], 0, 0)),
                pl.BlockSpec((OD, ED), lambda i, b: (0, 0)),
                pl.BlockSpec((1, OD), lambda i, b: (0, 0)),
            ],
            out_specs=pl.BlockSpec((BLK, OD2), lambda i, b: (i, 0)),
            scratch_shapes=[
                pltpu.VMEM((ED, D), jnp.bfloat16),
                pltpu.VMEM((OD, ED), jnp.bfloat16),
            ],
        ),
        out_shape=jax.ShapeDtypeStruct((xs.shape[0], OD2), jnp.int32),
        compiler_params=pltpu.CompilerParams(
            dimension_semantics=("arbitrary",)),
    )(bexp, xs, expert_W, expert_b.reshape(E, 1, ED), out_W,
      out_b.reshape(1, OD))


def _unpack_body(op_ref, out_ref):
    w = op_ref[...]
    out_ref[...] = jnp.concatenate([_unpack_lo(w), _unpack_hi(w)], axis=1)


def _unpack_out(opad):
    blk = 512
    return pl.pallas_call(
        _unpack_body,
        grid=(T // blk,),
        in_specs=[pl.BlockSpec((blk, OD2), lambda i: (i, 0))],
        out_specs=pl.BlockSpec((blk, OD), lambda i: (i, 0)),
        out_shape=jax.ShapeDtypeStruct((T, OD), jnp.float32),
    )(opad)


def _scatter_body(ys0_hbm, ys1_hbm, perm2_hbm, opad_hbm, idx_v, rows_v,
                  sem_i, sem_r, sem_s):
    cid = lax.axis_index("c")
    sid = lax.axis_index("s")
    wid = sid * NC + cid

    ci = [pltpu.async_copy(perm2_hbm.at[wid * CHUNKS + c], idx_v.at[c],
                           sem_i.at[c])
          for c in range(CHUNKS)]

    @pl.when(wid < NS)
    def _():
        cr = [pltpu.async_copy(ys0_hbm.at[wid, c], rows_v.at[c],
                               sem_r.at[c])
              for c in range(CHUNKS)]
        for d in cr:
            d.wait()

    @pl.when(wid >= NS)
    def _():
        cr = [pltpu.async_copy(ys1_hbm.at[wid - NS, c], rows_v.at[c],
                               sem_r.at[c])
              for c in range(CHUNKS)]
        for d in cr:
            d.wait()

    cs = [None] * CHUNKS
    for c in range(CHUNKS):
        ci[c].wait()
        cs[c] = pltpu.async_copy(rows_v.at[c], opad_hbm.at[idx_v.at[c]],
                                 sem_s.at[c])
    for c in range(CHUNKS):
        cs[c].wait()


@functools.cache
def _scatter():
    return pl.kernel(
        _scatter_body,
        out_type=jax.ShapeDtypeStruct((T + 8, OD2), jnp.int32),
        mesh=_sc_mesh(),
        compiler_params=pltpu.CompilerParams(needs_layout_passes=False),
        scratch_types=[
            pltpu.VMEM((CHUNKS, CR), jnp.int32),
            pltpu.VMEM((CHUNKS, CR, OD2), jnp.int32),
            pltpu.SemaphoreType.DMA((CHUNKS,)),
            pltpu.SemaphoreType.DMA((CHUNKS,)),
            pltpu.SemaphoreType.DMA((CHUNKS,)),
        ],
    )


def kernel(x, gate_W, gate_b, expert_W, expert_b, out_W, out_b):
    assert x.shape == (B, S, D)
    assert expert_W.shape == (E, ED, D)
    assert out_W.shape == (OD, ED)

    xf = x.reshape(T, D)
    assign, xp = _gate(xf, gate_W, gate_b)
    perm, bexp = _route_v2()(assign)
    half = CAP // 2
    nbh = NB // 2
    xs0 = _gather(2)(xp, perm[:half].reshape(NW * 2, CR))
    xs1 = _gather(2)(xp, perm[half:].reshape(NW * 2, CR))
    ys0 = _mm(bexp[:nbh], xs0.reshape(half, D2), expert_W, expert_b,
              out_W, out_b)
    ys1 = _mm(bexp[nbh:2 * nbh], xs1.reshape(half, D2), expert_W,
              expert_b, out_W, out_b)
    opad = _scatter()(ys0.reshape(NS, CHUNKS, CR, OD2),
                      ys1.reshape(NS, CHUNKS, CR, OD2),
                      perm.reshape(NW * CHUNKS, CR))
    return _unpack_out(opad).reshape(B, S, OD)
